# baseline (speedup 1.0000x reference)
import functools

import jax
import jax.numpy as jnp
from jax import lax
from jax.experimental import pallas as pl
from jax.experimental.pallas import tpu as pltpu
from jax.experimental.pallas import tpu_sc as plsc

NC = 2
NS = 16
NW = NC * NS
L = 16

D = 64
CHUNK = 128


@functools.lru_cache(maxsize=None)
def _make_sc_scorer(total):
    assert total % (NW * CHUNK) == 0
    per_w = total // NW
    n_chunks = per_w // CHUNK
    mesh = plsc.VectorSubcoreMesh(core_axis_name="c", subcore_axis_name="s")

    @functools.partial(
        pl.kernel,
        mesh=mesh,
        out_type=jax.ShapeDtypeStruct((total,), jnp.float32),
        compiler_params=pltpu.CompilerParams(
            needs_layout_passes=False, use_tc_tiling_on_sc=False),
        scratch_types=[
            pltpu.VMEM((CHUNK,), jnp.int32),
            pltpu.VMEM((CHUNK,), jnp.int32),
            pltpu.VMEM((CHUNK,), jnp.int32),
            pltpu.VMEM((CHUNK, D), jnp.float32),
            pltpu.VMEM((CHUNK, D), jnp.float32),
            pltpu.VMEM((CHUNK, D), jnp.float32),
            pltpu.VMEM((CHUNK, D), jnp.float32),
            pltpu.VMEM((CHUNK, D), jnp.float32),
            pltpu.VMEM((CHUNK, D), jnp.float32),
            pltpu.VMEM((CHUNK,), jnp.float32),
            pltpu.VMEM((L, L), jnp.float32),
            pltpu.SemaphoreType.DMA,
        ],
    )
    def scorer(h_hbm, t_hbm, r_hbm, eh_hbm, et_hbm, relh_hbm, relt_hbm,
               out_hbm, hi, ti, ri, hh, ht, th, tt, rh, rt, sv, stage, sem):
        wid = lax.axis_index("s") * NC + lax.axis_index("c")
        base = wid * per_w

        def chunk_body(c, carry):
            off = base + c * CHUNK
            pltpu.sync_copy(h_hbm.at[pl.ds(off, CHUNK)], hi)
            pltpu.sync_copy(t_hbm.at[pl.ds(off, CHUNK)], ti)
            pltpu.sync_copy(r_hbm.at[pl.ds(off, CHUNK)], ri)
            copies = [
                pltpu.async_copy(eh_hbm.at[hi], hh, sem),
                pltpu.async_copy(et_hbm.at[hi], ht, sem),
                pltpu.async_copy(eh_hbm.at[ti], th, sem),
                pltpu.async_copy(et_hbm.at[ti], tt, sem),
                pltpu.async_copy(relh_hbm.at[ri], rh, sem),
                pltpu.async_copy(relt_hbm.at[ri], rt, sem),
            ]
            for cp in copies:
                cp.wait()

            def group_body(g, carry2):
                i0 = g * L
                lanes = lax.iota(jnp.int32, L)
                for lane in range(L):
                    i = i0 + lane
                    acc = None
                    for j in range(D // L):
                        s = pl.ds(j * L, L)
                        term = hh[i, s] * rh[i, s] * tt[i, s] \
                             + th[i, s] * rt[i, s] * ht[i, s]
                        acc = term if acc is None else acc + term
                    col = jnp.full((L,), lane, jnp.int32)
                    plsc.store_scatter(stage, [lanes, col], acc)
                tot = stage[0, :]
                for k in range(1, L):
                    tot = tot + stage[k, :]
                sv[pl.ds(i0, L)] = 0.5 * tot
                return carry2

            lax.fori_loop(0, CHUNK // L, group_body, 0)
            pltpu.sync_copy(sv, out_hbm.at[pl.ds(off, CHUNK)])
            return carry

        lax.fori_loop(0, n_chunks, chunk_body, 0)

    return scorer


def kernel(pos_h, pos_r, pos_t, neg_h, neg_r, neg_t,
           entity_head, entity_tail, relation_head, relation_tail):
    b = pos_h.shape[0]
    h = jnp.concatenate([pos_h, neg_h])
    t = jnp.concatenate([pos_t, neg_t])
    r = jnp.concatenate([pos_r, neg_r])
    scorer = _make_sc_scorer(2 * b)
    out = scorer(h, t, r, entity_head, entity_tail, relation_head,
                 relation_tail)
    return out[:b], out[b:]


# speedup vs baseline: 1.1635x; 1.1635x over previous
import functools

import jax
import jax.numpy as jnp
from jax import lax
from jax.experimental import pallas as pl
from jax.experimental.pallas import tpu as pltpu
from jax.experimental.pallas import tpu_sc as plsc

NC = 2
NS = 16
NW = NC * NS
L = 16

D = 64
CHUNK = 64


@functools.lru_cache(maxsize=None)
def _make_sc_scorer(total):
    assert total % (NW * CHUNK) == 0
    per_w = total // NW
    n_chunks = per_w // CHUNK
    mesh = plsc.VectorSubcoreMesh(core_axis_name="c", subcore_axis_name="s")

    @functools.partial(
        pl.kernel,
        mesh=mesh,
        out_type=jax.ShapeDtypeStruct((total,), jnp.float32),
        compiler_params=pltpu.CompilerParams(needs_layout_passes=False),
        scratch_types=[
            pltpu.VMEM((CHUNK,), jnp.int32),
            pltpu.VMEM((CHUNK,), jnp.int32),
            pltpu.VMEM((CHUNK,), jnp.int32),
            pltpu.VMEM((CHUNK, D), jnp.float32),
            pltpu.VMEM((CHUNK, D), jnp.float32),
            pltpu.VMEM((CHUNK, D), jnp.float32),
            pltpu.VMEM((CHUNK, D), jnp.float32),
            pltpu.VMEM((CHUNK, D), jnp.float32),
            pltpu.VMEM((CHUNK, D), jnp.float32),
            pltpu.VMEM((per_w,), jnp.float32),
            pltpu.SemaphoreType.DMA,
        ],
    )
    def scorer(h_hbm, t_hbm, r_hbm, eh_hbm, et_hbm, relh_hbm, relt_hbm,
               out_hbm, hs, ts, rs, hh, ht, th, tt, rh, rt, sv, sem):
        wid = lax.axis_index("s") * NC + lax.axis_index("c")
        base = wid * per_w
        lanes = lax.iota(jnp.int32, L)

        def chunk_body(c, carry):
            off = base + c * CHUNK
            pltpu.sync_copy(h_hbm.at[pl.ds(off, CHUNK)], hs)
            pltpu.sync_copy(t_hbm.at[pl.ds(off, CHUNK)], ts)
            pltpu.sync_copy(r_hbm.at[pl.ds(off, CHUNK)], rs)

            def issue_body(g, carry2):
                i0 = g * L
                hvec = hs[pl.ds(i0, L)]
                tvec = ts[pl.ds(i0, L)]
                rvec = rs[pl.ds(i0, L)]
                for lane in range(L):
                    j = i0 + lane
                    h = hvec[lane]
                    t = tvec[lane]
                    r = rvec[lane]
                    pltpu.async_copy(eh_hbm.at[h], hh.at[j], sem)
                    pltpu.async_copy(et_hbm.at[h], ht.at[j], sem)
                    pltpu.async_copy(eh_hbm.at[t], th.at[j], sem)
                    pltpu.async_copy(et_hbm.at[t], tt.at[j], sem)
                    pltpu.async_copy(relh_hbm.at[r], rh.at[j], sem)
                    pltpu.async_copy(relt_hbm.at[r], rt.at[j], sem)
                return carry2

            lax.fori_loop(0, CHUNK // L, issue_body, 0)
            for buf in (hh, ht, th, tt, rh, rt):
                pltpu.make_async_copy(eh_hbm.at[pl.ds(0, CHUNK)], buf,
                                      sem).wait()

            def group_body(g, carry2):
                i0 = g * L
                rows = lanes + i0
                acc = jnp.zeros((L,), jnp.float32)
                for d in range(D):
                    dvec = jnp.full((L,), d, jnp.int32)
                    hhd = plsc.load_gather(hh, [rows, dvec])
                    htd = plsc.load_gather(ht, [rows, dvec])
                    thd = plsc.load_gather(th, [rows, dvec])
                    ttd = plsc.load_gather(tt, [rows, dvec])
                    rhd = plsc.load_gather(rh, [rows, dvec])
                    rtd = plsc.load_gather(rt, [rows, dvec])
                    acc = acc + (hhd * rhd * ttd + thd * rtd * htd)
                sv[pl.ds(c * CHUNK + i0, L)] = 0.5 * acc
                return carry2

            lax.fori_loop(0, CHUNK // L, group_body, 0)
            return carry

        lax.fori_loop(0, n_chunks, chunk_body, 0)
        pltpu.sync_copy(sv, out_hbm.at[pl.ds(base, per_w)])

    return scorer


def kernel(pos_h, pos_r, pos_t, neg_h, neg_r, neg_t,
           entity_head, entity_tail, relation_head, relation_tail):
    b = pos_h.shape[0]
    h = jnp.concatenate([pos_h, neg_h])
    t = jnp.concatenate([pos_t, neg_t])
    r = jnp.concatenate([pos_r, neg_r])
    scorer = _make_sc_scorer(2 * b)
    out = scorer(h, t, r, entity_head, entity_tail, relation_head,
                 relation_tail)
    return out[:b], out[b:]
